# baseline (speedup 1.0000x reference)
import functools

import jax
import jax.numpy as jnp
import numpy as np
from jax import lax
from jax.experimental import pallas as pl
from jax.experimental.pallas import tpu as pltpu
from jax.experimental.pallas import tpu_sc as plsc

_L = 2048
_D = 1024
_H = 16
_DK = 64
_NF = 1152
_TK = 7
_PREC = lax.Precision.HIGHEST
_PREC_PROJ = lax.Precision.DEFAULT

_INTERPRET = False


@functools.lru_cache(maxsize=1)
def _dft_tables():
    t = np.arange(_L, dtype=np.int64)
    f = np.arange(_NF, dtype=np.int64)
    ang = 2.0 * np.pi * ((f[:, None] * t[None, :]) % _L).astype(np.float64) / _L
    fc = np.cos(ang)
    fs = np.sin(ang)
    w = np.zeros((_NF, 1), dtype=np.float64)
    w[0, 0] = 1.0
    w[1:_L // 2, 0] = 2.0
    w[_L // 2, 0] = 1.0
    wc = (w / _L) * fc
    ws = (w / _L) * fs
    def _split(m):
        m32 = m.astype(np.float32)
        hi = m32.astype(jnp.bfloat16)
        lo = (m32 - hi.astype(np.float32)).astype(jnp.bfloat16)
        return hi, lo

    fch, fcl = _split(fc)
    fsh, fsl = _split(fs)
    return (fch, fcl, fsh, fsl,
            (w / _L).astype(np.float32),
            wc.T.copy().astype(np.float32), ws.T.copy().astype(np.float32))


def _proj3_body(qx, kx, vx, wq, wk, wv, oq, ok, ov):
    oq[...] = jnp.dot(qx[...], wq[...], precision=_PREC_PROJ,
                      preferred_element_type=jnp.float32)
    ok[...] = jnp.dot(kx[...], wk[...], precision=_PREC_PROJ,
                      preferred_element_type=jnp.float32)
    ov[...] = jnp.dot(vx[...], wv[...], precision=_PREC_PROJ,
                      preferred_element_type=jnp.float32)


def _split_bf16(x):
    hi = x.astype(jnp.bfloat16)
    lo = (x - hi.astype(jnp.float32)).astype(jnp.bfloat16)
    return hi, lo


def _dot3(ah, al, bh, bl, dn=None):
    if dn is None:
        op = lambda x, y: jnp.dot(x, y, precision=_PREC_PROJ,
                                  preferred_element_type=jnp.float32)
    else:
        op = lambda x, y: lax.dot_general(x, y, dn, precision=_PREC_PROJ,
                                          preferred_element_type=jnp.float32)
    return op(ah, bh) + op(ah, bl) + op(al, bh)


def _fwdpw_body(pq, pk, fch, fcl, fsh, fsl, wrow, g,
                prwh, prwl, piwh, piwl, pbr, pbi):
    b = pl.program_id(1)
    d2 = pl.program_id(2)
    fchv, fclv, fshv, fslv = fch[...], fcl[...], fsh[...], fsl[...]
    pqh, pql = _split_bf16(pq[0])
    pkh, pkl = _split_bf16(pk[0])
    qc = _dot3(fchv, fclv, pqh, pql)
    qs = _dot3(fshv, fslv, pqh, pql)
    kc = _dot3(fchv, fclv, pkh, pkl)
    ks = _dot3(fshv, fslv, pkh, pkl)
    prv = qc * kc + qs * ks
    piv = qc * ks - qs * kc
    wv = wrow[...]
    ph, plo = _split_bf16(prv * wv)
    prwh[0] = ph.T
    prwl[0] = plo.T
    ph, plo = _split_bf16(piv * wv)
    piwh[0] = ph.T
    piwl[0] = plo.T
    gv = g[...]
    pbr_c = jnp.dot(prv, gv, precision=_PREC, preferred_element_type=jnp.float32)
    pbi_c = jnp.dot(piv, gv, precision=_PREC, preferred_element_type=jnp.float32)
    first = jnp.logical_and(b == 0, d2 == 0)

    @pl.when(first)
    def _():
        pbr[...] = pbr_c
        pbi[...] = pbi_c

    @pl.when(jnp.logical_not(first))
    def _():
        pbr[...] += pbr_c
        pbi[...] += pbi_c


def _inv2_body(prwh, prwl, piwh, piwl, fch, fcl, fsh, fsl, o):
    o[0] = (_dot3(prwh[0], prwl[0], fch[...], fcl[...])
            - _dot3(piwh[0], piwl[0], fsh[...], fsl[...]))


def _mv_body(wct, wst, pbr, pbi, mvt64, mcol):
    mv = (jnp.dot(wct[...], pbr[...], precision=_PREC,
                  preferred_element_type=jnp.float32)
          - jnp.dot(wst[...], pbi[...], precision=_PREC,
                    preferred_element_type=jnp.float32))
    mvt64[...] = mv
    mcol[...] = jnp.sum(mv[:, :_DK], axis=1, keepdims=True) * (1.0 / _DK)


def _agg_body(sh, pv, smw, wot, bo2, o):
    h = pl.program_id(1)
    pvv = pv[0]
    acc = None
    for i in range(_TK):
        s = sh[i]
        shift = jnp.where(s == 0, 0, _L - s)
        term = pltpu.roll(pvv, shift, 0) * smw[i:i + 1, :]
        acc = term if acc is None else acc + term
    contrib = jnp.dot(acc, wot[...], precision=_PREC_PROJ,
                      preferred_element_type=jnp.float32)

    @pl.when(h == 0)
    def _():
        o[0] = contrib + bo2[...]

    @pl.when(h > 0)
    def _():
        o[0] += contrib


def _make_topk_sc():
    mesh = plsc.VectorSubcoreMesh(core_axis_name="c", subcore_axis_name="s")

    @functools.partial(
        pl.kernel, mesh=mesh,
        out_type=(jax.ShapeDtypeStruct((16,), jnp.int32),
                  jax.ShapeDtypeStruct((512,), jnp.float32)),
        scratch_types=[pltpu.VMEM((_L,), jnp.float32),
                       pltpu.VMEM((16,), jnp.int32),
                       pltpu.VMEM((16, 128), jnp.float32),
                       pltpu.VMEM((512,), jnp.float32),
                       pltpu.SemaphoreType.DMA],
    )
    def topk_sc(m_hbm, mvt_hbm, idx_out, smt_out, m_v, idx16_v,
                rows_v, smt_v, sem):
        cid = lax.axis_index("c")
        sid = lax.axis_index("s")

        @pl.when(jnp.logical_and(cid == 0, sid == 0))
        def _():
            pltpu.sync_copy(m_hbm, m_v)
            lane = lax.iota(jnp.int32, 16)
            idxvec = jnp.zeros((16,), jnp.int32)
            chosen = []
            for p in range(_TK):
                def body(j, carry, _chosen=tuple(chosen)):
                    bv, bi = carry
                    v = m_v[pl.ds(j * 16, 16)]
                    iv = lane + j * 16
                    for gprev in _chosen:
                        v = jnp.where(iv == gprev, -3e38, v)
                    better = v > bv
                    return (jnp.where(better, v, bv),
                            jnp.where(better, iv, bi))

                bv, bi = lax.fori_loop(
                    0, _L // 16, body,
                    (jnp.full((16,), -3e38, jnp.float32),
                     jnp.zeros((16,), jnp.int32)))
                bestv = bv[0]
                bestg = bi[0]
                for q in range(1, 16):
                    v = bv[q]
                    gi = bi[q]
                    take = jnp.logical_or(
                        v > bestv,
                        jnp.logical_and(v == bestv, gi < bestg))
                    bestv = jnp.where(take, v, bestv)
                    bestg = jnp.where(take, gi, bestg)
                g = bestg
                idxvec = jnp.where(lane == p, g, idxvec)
                chosen.append(g)
            idx16_v[...] = idxvec
            pltpu.async_copy(mvt_hbm.at[idx16_v], rows_v, sem).wait()
            for j in range(_DK // 16):
                cols = [rows_v[p, pl.ds(j * 16, 16)] for p in range(_TK)]
                mxc = cols[0]
                for p in range(1, _TK):
                    mxc = jnp.maximum(mxc, cols[p])
                exps = [jnp.exp(c - mxc) for c in cols]
                tot = exps[0]
                for p in range(1, _TK):
                    tot = tot + exps[p]
                inv = 1.0 / tot
                for p in range(_TK):
                    smt_v[pl.ds(p * _DK + j * 16, 16)] = exps[p] * inv
                smt_v[pl.ds(7 * _DK + j * 16, 16)] = jnp.zeros((16,), jnp.float32)
            pltpu.sync_copy(idx16_v, idx_out)
            pltpu.sync_copy(smt_v, smt_out)

    return topk_sc


def kernel(queries, keys, values, Wq, Wk, Wv, Wo, bo):
    B, L, D = queries.shape
    fch, fcl, fsh, fsl, wrow, wct, wst = _dft_tables()
    f32 = jnp.float32
    bf16 = jnp.bfloat16

    BLK = 512
    nblk = (B * L) // BLK
    proj = pl.pallas_call(
        _proj3_body,
        grid=(nblk,),
        in_specs=[pl.BlockSpec((BLK, D), lambda i: (i, 0))] * 3
        + [pl.BlockSpec((D, D), lambda i: (0, 0))] * 3,
        out_specs=[pl.BlockSpec((BLK, D), lambda i: (i, 0))] * 3,
        out_shape=[jax.ShapeDtypeStruct((B * L, D), f32)] * 3,
        interpret=_INTERPRET,
    )
    pq, pk, pv = proj(queries.reshape(B * L, D), keys.reshape(B * L, D),
                      values.reshape(B * L, D), Wq.T, Wk.T, Wv.T)
    pq = pq.reshape(B, L, D)
    pk = pk.reshape(B, L, D)
    pv = pv.reshape(B, L, D)

    FB = 384
    DB = 256
    gfull = np.zeros((D, 128), dtype=np.float32)
    hc = np.arange(D)
    gfull[hc, hc % _DK] = 1.0 / (B * _H)
    fwdpw = pl.pallas_call(
        _fwdpw_body,
        grid=(_NF // FB, B, D // DB),
        in_specs=[pl.BlockSpec((1, L, DB), lambda j, b, d: (b, 0, d))] * 2
        + [pl.BlockSpec((FB, L), lambda j, b, d: (j, 0))] * 4
        + [pl.BlockSpec((FB, 1), lambda j, b, d: (j, 0)),
           pl.BlockSpec((DB, 128), lambda j, b, d: (d, 0))],
        out_specs=[pl.BlockSpec((1, DB, FB), lambda j, b, d: (b, d, j))] * 4
        + [pl.BlockSpec((FB, 128), lambda j, b, d: (j, 0))] * 2,
        out_shape=[jax.ShapeDtypeStruct((B, D, _NF), bf16)] * 4
        + [jax.ShapeDtypeStruct((_NF, 128), f32)] * 2,
        interpret=_INTERPRET,
    )
    prwh, prwl, piwh, piwl, pbr, pbi = fwdpw(
        pq, pk, jnp.asarray(fch), jnp.asarray(fcl),
        jnp.asarray(fsh), jnp.asarray(fsl),
        jnp.asarray(wrow.reshape(_NF, 1)), jnp.asarray(gfull))

    TB = 512
    inv2 = pl.pallas_call(
        _inv2_body,
        grid=(B, L // TB),
        in_specs=[pl.BlockSpec((1, D, _NF), lambda b, j: (b, 0, 0))] * 4
        + [pl.BlockSpec((_NF, TB), lambda b, j: (0, j))] * 4,
        out_specs=pl.BlockSpec((1, D, TB), lambda b, j: (b, 0, j)),
        out_shape=jax.ShapeDtypeStruct((B, D, L), f32),
        interpret=_INTERPRET,
    )
    corr = inv2(prwh, prwl, piwh, piwl,
                jnp.asarray(fch), jnp.asarray(fcl),
                jnp.asarray(fsh), jnp.asarray(fsl))

    mv = pl.pallas_call(
        _mv_body,
        grid=(1,),
        in_specs=[pl.BlockSpec((L, _NF), lambda i: (0, 0))] * 2
        + [pl.BlockSpec((_NF, 128), lambda i: (0, 0))] * 2,
        out_specs=[pl.BlockSpec((L, 128), lambda i: (0, 0)),
                   pl.BlockSpec((L, 1), lambda i: (0, 0))],
        out_shape=[jax.ShapeDtypeStruct((L, 128), f32),
                   jax.ShapeDtypeStruct((L, 1), f32)],
        interpret=_INTERPRET,
    )
    mvt64, mcol = mv(jnp.asarray(wct), jnp.asarray(wst), pbr, pbi)

    idx16, smt_flat = _make_topk_sc()(mcol.reshape(L), mvt64)
    smw = jnp.tile(smt_flat.reshape(8, _DK), (1, _H))

    HCB2 = 256
    agg = pl.pallas_call(
        _agg_body,
        grid=(B, D // HCB2),
        in_specs=[pl.BlockSpec(memory_space=pltpu.SMEM),
                  pl.BlockSpec((1, L, HCB2), lambda b, h: (b, 0, h)),
                  pl.BlockSpec((8, HCB2), lambda b, h: (0, h)),
                  pl.BlockSpec((HCB2, D), lambda b, h: (h, 0)),
                  pl.BlockSpec((1, D), lambda b, h: (0, 0))],
        out_specs=pl.BlockSpec((1, L, D), lambda b, h: (b, 0, 0)),
        out_shape=jax.ShapeDtypeStruct((B, L, D), f32),
        interpret=_INTERPRET,
    )
    out = agg(idx16, pv, smw, Wo.T, bo.reshape(1, D))

    return out, corr.reshape(B, _H, _DK, L)


# speedup vs baseline: 1.1304x; 1.1304x over previous
import functools

import jax
import jax.numpy as jnp
import numpy as np
from jax import lax
from jax.experimental import pallas as pl
from jax.experimental.pallas import tpu as pltpu
from jax.experimental.pallas import tpu_sc as plsc

_L = 2048
_D = 1024
_H = 16
_DK = 64
_NF = 1152
_TK = 7
_PREC = lax.Precision.HIGHEST
_PREC_PROJ = lax.Precision.DEFAULT

_INTERPRET = False


@functools.lru_cache(maxsize=1)
def _dft_tables():
    t = np.arange(_L, dtype=np.int64)
    f = np.arange(_NF, dtype=np.int64)
    ang = 2.0 * np.pi * ((f[:, None] * t[None, :]) % _L).astype(np.float64) / _L
    fc = np.cos(ang)
    fs = np.sin(ang)
    w = np.zeros((_NF, 1), dtype=np.float64)
    w[0, 0] = 1.0
    w[1:_L // 2, 0] = 2.0
    w[_L // 2, 0] = 1.0
    wc = (w / _L) * fc
    ws = (w / _L) * fs
    def _split(m):
        m32 = m.astype(np.float32)
        hi = m32.astype(jnp.bfloat16)
        lo = (m32 - hi.astype(np.float32)).astype(jnp.bfloat16)
        return hi, lo

    fch, fcl = _split(fc)
    fsh, fsl = _split(fs)
    return (fch, fcl, fsh, fsl,
            (w / _L).astype(np.float32),
            wc.T.copy().astype(np.float32), ws.T.copy().astype(np.float32))


def _proj3_body(qx, kx, vx, wq, wk, wv, oq, ok, ov):
    oq[...] = jnp.dot(qx[...], wq[...], precision=_PREC_PROJ,
                      preferred_element_type=jnp.float32)
    ok[...] = jnp.dot(kx[...], wk[...], precision=_PREC_PROJ,
                      preferred_element_type=jnp.float32)
    ov[...] = jnp.dot(vx[...], wv[...], precision=_PREC_PROJ,
                      preferred_element_type=jnp.float32)


def _split_bf16(x):
    hi = x.astype(jnp.bfloat16)
    lo = (x - hi.astype(jnp.float32)).astype(jnp.bfloat16)
    return hi, lo


def _dot3(ah, al, bh, bl, dn=None):
    if dn is None:
        op = lambda x, y: jnp.dot(x, y, precision=_PREC_PROJ,
                                  preferred_element_type=jnp.float32)
    else:
        op = lambda x, y: lax.dot_general(x, y, dn, precision=_PREC_PROJ,
                                          preferred_element_type=jnp.float32)
    return op(ah, bh) + op(ah, bl) + op(al, bh)


def _fwdpw_body(pq, pk, fch, fcl, fsh, fsl, wrow, g,
                prwh, prwl, piwh, piwl, pbr, pbi):
    b = pl.program_id(1)
    d2 = pl.program_id(2)
    fchv, fclv, fshv, fslv = fch[...], fcl[...], fsh[...], fsl[...]
    pqh, pql = _split_bf16(pq[0])
    pkh, pkl = _split_bf16(pk[0])
    qc = _dot3(fchv, fclv, pqh, pql)
    qs = _dot3(fshv, fslv, pqh, pql)
    kc = _dot3(fchv, fclv, pkh, pkl)
    ks = _dot3(fshv, fslv, pkh, pkl)
    prv = qc * kc + qs * ks
    piv = qc * ks - qs * kc
    wv = wrow[...]
    ph, plo = _split_bf16(prv * wv)
    prwh[0] = ph.T
    prwl[0] = plo.T
    ph, plo = _split_bf16(piv * wv)
    piwh[0] = ph.T
    piwl[0] = plo.T
    gv = g[...]
    pbr_c = jnp.dot(prv, gv, precision=_PREC, preferred_element_type=jnp.float32)
    pbi_c = jnp.dot(piv, gv, precision=_PREC, preferred_element_type=jnp.float32)
    first = jnp.logical_and(b == 0, d2 == 0)

    @pl.when(first)
    def _():
        pbr[...] = pbr_c
        pbi[...] = pbi_c

    @pl.when(jnp.logical_not(first))
    def _():
        pbr[...] += pbr_c
        pbi[...] += pbi_c


def _inv2_body(prwh, prwl, piwh, piwl, fch, fcl, fsh, fsl, o):
    o[0] = (_dot3(prwh[0], prwl[0], fch[...], fcl[...])
            - _dot3(piwh[0], piwl[0], fsh[...], fsl[...]))


def _mv_body(wct, wst, pbr, pbi, mvt64, mcol):
    mv = (jnp.dot(wct[...], pbr[...], precision=_PREC,
                  preferred_element_type=jnp.float32)
          - jnp.dot(wst[...], pbi[...], precision=_PREC,
                    preferred_element_type=jnp.float32))
    mvt64[...] = mv
    mcol[...] = jnp.sum(mv[:, :_DK], axis=1, keepdims=True) * (1.0 / _DK)


def _agg_body(sh, pv, smw, wot, bo2, o):
    h = pl.program_id(1)
    pvv = pv[0].astype(jnp.bfloat16)
    acc = None
    for i in range(_TK):
        s = sh[i]
        shift = jnp.where(s == 0, 0, _L - s)
        term = pltpu.roll(pvv, shift, 0).astype(jnp.float32) * smw[i:i + 1, :]
        acc = term if acc is None else acc + term
    contrib = jnp.dot(acc, wot[...], precision=_PREC_PROJ,
                      preferred_element_type=jnp.float32)

    @pl.when(h == 0)
    def _():
        o[0] = contrib + bo2[...]

    @pl.when(h > 0)
    def _():
        o[0] += contrib


def _make_topk_sc():
    mesh = plsc.VectorSubcoreMesh(core_axis_name="c", subcore_axis_name="s")

    @functools.partial(
        pl.kernel, mesh=mesh,
        out_type=(jax.ShapeDtypeStruct((16,), jnp.int32),
                  jax.ShapeDtypeStruct((512,), jnp.float32)),
        scratch_types=[pltpu.VMEM((_L,), jnp.float32),
                       pltpu.VMEM((16,), jnp.int32),
                       pltpu.VMEM((16, 128), jnp.float32),
                       pltpu.VMEM((512,), jnp.float32),
                       pltpu.SemaphoreType.DMA],
    )
    def topk_sc(m_hbm, mvt_hbm, idx_out, smt_out, m_v, idx16_v,
                rows_v, smt_v, sem):
        cid = lax.axis_index("c")
        sid = lax.axis_index("s")

        @pl.when(jnp.logical_and(cid == 0, sid == 0))
        def _():
            pltpu.sync_copy(m_hbm, m_v)
            lane = lax.iota(jnp.int32, 16)
            idxvec = jnp.zeros((16,), jnp.int32)
            chosen = []
            for p in range(_TK):
                def body(j, carry, _chosen=tuple(chosen)):
                    bv, bi = carry
                    v = m_v[pl.ds(j * 16, 16)]
                    iv = lane + j * 16
                    for gprev in _chosen:
                        v = jnp.where(iv == gprev, -3e38, v)
                    better = v > bv
                    return (jnp.where(better, v, bv),
                            jnp.where(better, iv, bi))

                bv, bi = lax.fori_loop(
                    0, _L // 16, body,
                    (jnp.full((16,), -3e38, jnp.float32),
                     jnp.zeros((16,), jnp.int32)))
                bestv = bv[0]
                bestg = bi[0]
                for q in range(1, 16):
                    v = bv[q]
                    gi = bi[q]
                    take = jnp.logical_or(
                        v > bestv,
                        jnp.logical_and(v == bestv, gi < bestg))
                    bestv = jnp.where(take, v, bestv)
                    bestg = jnp.where(take, gi, bestg)
                g = bestg
                idxvec = jnp.where(lane == p, g, idxvec)
                chosen.append(g)
            idx16_v[...] = idxvec
            pltpu.async_copy(mvt_hbm.at[idx16_v], rows_v, sem).wait()
            for j in range(_DK // 16):
                cols = [rows_v[p, pl.ds(j * 16, 16)] for p in range(_TK)]
                mxc = cols[0]
                for p in range(1, _TK):
                    mxc = jnp.maximum(mxc, cols[p])
                exps = [jnp.exp(c - mxc) for c in cols]
                tot = exps[0]
                for p in range(1, _TK):
                    tot = tot + exps[p]
                inv = 1.0 / tot
                for p in range(_TK):
                    smt_v[pl.ds(p * _DK + j * 16, 16)] = exps[p] * inv
                smt_v[pl.ds(7 * _DK + j * 16, 16)] = jnp.zeros((16,), jnp.float32)
            pltpu.sync_copy(idx16_v, idx_out)
            pltpu.sync_copy(smt_v, smt_out)

    return topk_sc


def kernel(queries, keys, values, Wq, Wk, Wv, Wo, bo):
    B, L, D = queries.shape
    fch, fcl, fsh, fsl, wrow, wct, wst = _dft_tables()
    f32 = jnp.float32
    bf16 = jnp.bfloat16

    BLK = 512
    nblk = (B * L) // BLK
    proj = pl.pallas_call(
        _proj3_body,
        grid=(nblk,),
        in_specs=[pl.BlockSpec((BLK, D), lambda i: (i, 0))] * 3
        + [pl.BlockSpec((D, D), lambda i: (0, 0))] * 3,
        out_specs=[pl.BlockSpec((BLK, D), lambda i: (i, 0))] * 3,
        out_shape=[jax.ShapeDtypeStruct((B * L, D), f32)] * 3,
        interpret=_INTERPRET,
    )
    pq, pk, pv = proj(queries.reshape(B * L, D), keys.reshape(B * L, D),
                      values.reshape(B * L, D), Wq.T, Wk.T, Wv.T)
    pq = pq.reshape(B, L, D)
    pk = pk.reshape(B, L, D)
    pv = pv.reshape(B, L, D)

    FB = 384
    DB = 256
    gfull = np.zeros((D, 128), dtype=np.float32)
    hc = np.arange(D)
    gfull[hc, hc % _DK] = 1.0 / (B * _H)
    fwdpw = pl.pallas_call(
        _fwdpw_body,
        grid=(_NF // FB, B, D // DB),
        in_specs=[pl.BlockSpec((1, L, DB), lambda j, b, d: (b, 0, d))] * 2
        + [pl.BlockSpec((FB, L), lambda j, b, d: (j, 0))] * 4
        + [pl.BlockSpec((FB, 1), lambda j, b, d: (j, 0)),
           pl.BlockSpec((DB, 128), lambda j, b, d: (d, 0))],
        out_specs=[pl.BlockSpec((1, DB, FB), lambda j, b, d: (b, d, j))] * 4
        + [pl.BlockSpec((FB, 128), lambda j, b, d: (j, 0))] * 2,
        out_shape=[jax.ShapeDtypeStruct((B, D, _NF), bf16)] * 4
        + [jax.ShapeDtypeStruct((_NF, 128), f32)] * 2,
        interpret=_INTERPRET,
    )
    prwh, prwl, piwh, piwl, pbr, pbi = fwdpw(
        pq, pk, jnp.asarray(fch), jnp.asarray(fcl),
        jnp.asarray(fsh), jnp.asarray(fsl),
        jnp.asarray(wrow.reshape(_NF, 1)), jnp.asarray(gfull))

    TB = 512
    inv2 = pl.pallas_call(
        _inv2_body,
        grid=(B, L // TB),
        in_specs=[pl.BlockSpec((1, D, _NF), lambda b, j: (b, 0, 0))] * 4
        + [pl.BlockSpec((_NF, TB), lambda b, j: (0, j))] * 4,
        out_specs=pl.BlockSpec((1, D, TB), lambda b, j: (b, 0, j)),
        out_shape=jax.ShapeDtypeStruct((B, D, L), f32),
        interpret=_INTERPRET,
    )
    corr = inv2(prwh, prwl, piwh, piwl,
                jnp.asarray(fch), jnp.asarray(fcl),
                jnp.asarray(fsh), jnp.asarray(fsl))

    mv = pl.pallas_call(
        _mv_body,
        grid=(1,),
        in_specs=[pl.BlockSpec((L, _NF), lambda i: (0, 0))] * 2
        + [pl.BlockSpec((_NF, 128), lambda i: (0, 0))] * 2,
        out_specs=[pl.BlockSpec((L, 128), lambda i: (0, 0)),
                   pl.BlockSpec((L, 1), lambda i: (0, 0))],
        out_shape=[jax.ShapeDtypeStruct((L, 128), f32),
                   jax.ShapeDtypeStruct((L, 1), f32)],
        interpret=_INTERPRET,
    )
    mvt64, mcol = mv(jnp.asarray(wct), jnp.asarray(wst), pbr, pbi)

    idx16, smt_flat = _make_topk_sc()(mcol.reshape(L), mvt64)
    smw = jnp.tile(smt_flat.reshape(8, _DK), (1, _H))

    HCB2 = 256
    agg = pl.pallas_call(
        _agg_body,
        grid=(B, D // HCB2),
        in_specs=[pl.BlockSpec(memory_space=pltpu.SMEM),
                  pl.BlockSpec((1, L, HCB2), lambda b, h: (b, 0, h)),
                  pl.BlockSpec((8, HCB2), lambda b, h: (0, h)),
                  pl.BlockSpec((HCB2, D), lambda b, h: (h, 0)),
                  pl.BlockSpec((1, D), lambda b, h: (0, 0))],
        out_specs=pl.BlockSpec((1, L, D), lambda b, h: (b, 0, 0)),
        out_shape=jax.ShapeDtypeStruct((B, L, D), f32),
        interpret=_INTERPRET,
    )
    out = agg(idx16, pv, smw, Wo.T, bo.reshape(1, D))

    return out, corr.reshape(B, _H, _DK, L)


# speedup vs baseline: 1.2595x; 1.1142x over previous
import functools

import jax
import jax.numpy as jnp
import numpy as np
from jax import lax
from jax.experimental import pallas as pl
from jax.experimental.pallas import tpu as pltpu
from jax.experimental.pallas import tpu_sc as plsc

_L = 2048
_D = 1024
_H = 16
_DK = 64
_NF = 1152
_TK = 7
_PREC = lax.Precision.HIGHEST
_PREC_PROJ = lax.Precision.DEFAULT

_INTERPRET = False


@functools.lru_cache(maxsize=1)
def _dft_tables():
    t = np.arange(_L, dtype=np.int64)
    f = np.arange(_NF, dtype=np.int64)
    ang = 2.0 * np.pi * ((f[:, None] * t[None, :]) % _L).astype(np.float64) / _L
    fc = np.cos(ang)
    fs = np.sin(ang)
    w = np.zeros((_NF, 1), dtype=np.float64)
    w[0, 0] = 1.0
    w[1:_L // 2, 0] = 2.0
    w[_L // 2, 0] = 1.0
    wc = (w / _L) * fc
    ws = (w / _L) * fs
    def _split(m):
        m32 = m.astype(np.float32)
        hi = m32.astype(jnp.bfloat16)
        lo = (m32 - hi.astype(np.float32)).astype(jnp.bfloat16)
        return hi, lo

    fch, fcl = _split(fc)
    fsh, fsl = _split(fs)
    return (fch, fcl, fsh, fsl,
            (w / _L).astype(np.float32),
            wc.T.copy().astype(np.float32), ws.T.copy().astype(np.float32))


def _proj3_body(qx, kx, vx, wq, wk, wv, oq, ok, ov):
    oq[...] = jnp.dot(qx[...], wq[...], precision=_PREC_PROJ,
                      preferred_element_type=jnp.float32)
    ok[...] = jnp.dot(kx[...], wk[...], precision=_PREC_PROJ,
                      preferred_element_type=jnp.float32)
    ov[...] = jnp.dot(vx[...], wv[...], precision=_PREC_PROJ,
                      preferred_element_type=jnp.float32)


def _split_bf16(x):
    hi = x.astype(jnp.bfloat16)
    lo = (x - hi.astype(jnp.float32)).astype(jnp.bfloat16)
    return hi, lo


def _dot3(ah, al, bh, bl, dn=None):
    if dn is None:
        op = lambda x, y: jnp.dot(x, y, precision=_PREC_PROJ,
                                  preferred_element_type=jnp.float32)
    else:
        op = lambda x, y: lax.dot_general(x, y, dn, precision=_PREC_PROJ,
                                          preferred_element_type=jnp.float32)
    return op(ah, bh) + op(ah, bl) + op(al, bh)


def _fwdpw_body(pq, pk, fch, fcl, fsh, fsl, wrow, g,
                prwh, prwl, piwh, piwl, pbr, pbi):
    b = pl.program_id(1)
    d2 = pl.program_id(2)
    fchv, fclv, fshv, fslv = fch[...], fcl[...], fsh[...], fsl[...]
    pqh = pq[0].astype(jnp.bfloat16)
    pkh = pk[0].astype(jnp.bfloat16)

    def dot2(mh, ml, x):
        return (jnp.dot(mh, x, precision=_PREC_PROJ,
                        preferred_element_type=jnp.float32)
                + jnp.dot(ml, x, precision=_PREC_PROJ,
                          preferred_element_type=jnp.float32))

    qc = dot2(fchv, fclv, pqh)
    qs = dot2(fshv, fslv, pqh)
    kc = dot2(fchv, fclv, pkh)
    ks = dot2(fshv, fslv, pkh)
    prv = qc * kc + qs * ks
    piv = qc * ks - qs * kc
    wv = wrow[...]
    ph, plo = _split_bf16(prv * wv)
    prwh[0] = ph.T
    prwl[0] = plo.T
    ph, plo = _split_bf16(piv * wv)
    piwh[0] = ph.T
    piwl[0] = plo.T
    gv = g[...]
    pbr_c = jnp.dot(prv, gv, precision=_PREC, preferred_element_type=jnp.float32)
    pbi_c = jnp.dot(piv, gv, precision=_PREC, preferred_element_type=jnp.float32)
    first = jnp.logical_and(b == 0, d2 == 0)

    @pl.when(first)
    def _():
        pbr[...] = pbr_c
        pbi[...] = pbi_c

    @pl.when(jnp.logical_not(first))
    def _():
        pbr[...] += pbr_c
        pbi[...] += pbi_c


def _inv2_body(prwh, prwl, piwh, piwl, fch, fcl, fsh, fsl, o):
    o[0] = (_dot3(prwh[0], prwl[0], fch[...], fcl[...])
            - _dot3(piwh[0], piwl[0], fsh[...], fsl[...]))


def _mv_body(wct, wst, pbr, pbi, mvt64, mcol):
    mv = (jnp.dot(wct[...], pbr[...], precision=_PREC,
                  preferred_element_type=jnp.float32)
          - jnp.dot(wst[...], pbi[...], precision=_PREC,
                    preferred_element_type=jnp.float32))
    mvt64[...] = mv
    mcol[...] = jnp.sum(mv[:, :_DK], axis=1, keepdims=True) * (1.0 / _DK)


def _agg_body(sh, pv, smw, wot, bo2, o):
    h = pl.program_id(1)
    pvv = pv[0].astype(jnp.bfloat16)
    acc = None
    for i in range(_TK):
        s = sh[i]
        shift = jnp.where(s == 0, 0, _L - s)
        term = pltpu.roll(pvv, shift, 0).astype(jnp.float32) * smw[i:i + 1, :]
        acc = term if acc is None else acc + term
    contrib = jnp.dot(acc, wot[...], precision=_PREC_PROJ,
                      preferred_element_type=jnp.float32)

    @pl.when(h == 0)
    def _():
        o[0] = contrib + bo2[...]

    @pl.when(h > 0)
    def _():
        o[0] += contrib


def _make_topk_sc():
    mesh = plsc.VectorSubcoreMesh(core_axis_name="c", subcore_axis_name="s")

    @functools.partial(
        pl.kernel, mesh=mesh,
        out_type=(jax.ShapeDtypeStruct((16,), jnp.int32),
                  jax.ShapeDtypeStruct((512,), jnp.float32)),
        scratch_types=[pltpu.VMEM((_L,), jnp.float32),
                       pltpu.VMEM((16,), jnp.int32),
                       pltpu.VMEM((16, 128), jnp.float32),
                       pltpu.VMEM((512,), jnp.float32),
                       pltpu.SemaphoreType.DMA],
    )
    def topk_sc(m_hbm, mvt_hbm, idx_out, smt_out, m_v, idx16_v,
                rows_v, smt_v, sem):
        cid = lax.axis_index("c")
        sid = lax.axis_index("s")

        @pl.when(jnp.logical_and(cid == 0, sid == 0))
        def _():
            pltpu.sync_copy(m_hbm, m_v)
            lane = lax.iota(jnp.int32, 16)
            idxvec = jnp.zeros((16,), jnp.int32)
            chosen = []
            for p in range(_TK):
                def body(j, carry, _chosen=tuple(chosen)):
                    bv, bi = carry
                    v = m_v[pl.ds(j * 16, 16)]
                    iv = lane + j * 16
                    for gprev in _chosen:
                        v = jnp.where(iv == gprev, -3e38, v)
                    better = v > bv
                    return (jnp.where(better, v, bv),
                            jnp.where(better, iv, bi))

                bv, bi = lax.fori_loop(
                    0, _L // 16, body,
                    (jnp.full((16,), -3e38, jnp.float32),
                     jnp.zeros((16,), jnp.int32)))
                bestv = bv[0]
                bestg = bi[0]
                for q in range(1, 16):
                    v = bv[q]
                    gi = bi[q]
                    take = jnp.logical_or(
                        v > bestv,
                        jnp.logical_and(v == bestv, gi < bestg))
                    bestv = jnp.where(take, v, bestv)
                    bestg = jnp.where(take, gi, bestg)
                g = bestg
                idxvec = jnp.where(lane == p, g, idxvec)
                chosen.append(g)
            idx16_v[...] = idxvec
            pltpu.async_copy(mvt_hbm.at[idx16_v], rows_v, sem).wait()
            for j in range(_DK // 16):
                cols = [rows_v[p, pl.ds(j * 16, 16)] for p in range(_TK)]
                mxc = cols[0]
                for p in range(1, _TK):
                    mxc = jnp.maximum(mxc, cols[p])
                exps = [jnp.exp(c - mxc) for c in cols]
                tot = exps[0]
                for p in range(1, _TK):
                    tot = tot + exps[p]
                inv = 1.0 / tot
                for p in range(_TK):
                    smt_v[pl.ds(p * _DK + j * 16, 16)] = exps[p] * inv
                smt_v[pl.ds(7 * _DK + j * 16, 16)] = jnp.zeros((16,), jnp.float32)
            pltpu.sync_copy(idx16_v, idx_out)
            pltpu.sync_copy(smt_v, smt_out)

    return topk_sc


def kernel(queries, keys, values, Wq, Wk, Wv, Wo, bo):
    B, L, D = queries.shape
    fch, fcl, fsh, fsl, wrow, wct, wst = _dft_tables()
    f32 = jnp.float32
    bf16 = jnp.bfloat16

    BLK = 512
    nblk = (B * L) // BLK
    proj = pl.pallas_call(
        _proj3_body,
        grid=(nblk,),
        in_specs=[pl.BlockSpec((BLK, D), lambda i: (i, 0))] * 3
        + [pl.BlockSpec((D, D), lambda i: (0, 0))] * 3,
        out_specs=[pl.BlockSpec((BLK, D), lambda i: (i, 0))] * 3,
        out_shape=[jax.ShapeDtypeStruct((B * L, D), f32)] * 3,
        interpret=_INTERPRET,
    )
    pq, pk, pv = proj(queries.reshape(B * L, D), keys.reshape(B * L, D),
                      values.reshape(B * L, D), Wq.T, Wk.T, Wv.T)
    pq = pq.reshape(B, L, D)
    pk = pk.reshape(B, L, D)
    pv = pv.reshape(B, L, D)

    FB = 384
    DB = 256
    gfull = np.zeros((D, 128), dtype=np.float32)
    hc = np.arange(D)
    gfull[hc, hc % _DK] = 1.0 / (B * _H)
    fwdpw = pl.pallas_call(
        _fwdpw_body,
        grid=(_NF // FB, B, D // DB),
        in_specs=[pl.BlockSpec((1, L, DB), lambda j, b, d: (b, 0, d))] * 2
        + [pl.BlockSpec((FB, L), lambda j, b, d: (j, 0))] * 4
        + [pl.BlockSpec((FB, 1), lambda j, b, d: (j, 0)),
           pl.BlockSpec((DB, 128), lambda j, b, d: (d, 0))],
        out_specs=[pl.BlockSpec((1, DB, FB), lambda j, b, d: (b, d, j))] * 4
        + [pl.BlockSpec((FB, 128), lambda j, b, d: (j, 0))] * 2,
        out_shape=[jax.ShapeDtypeStruct((B, D, _NF), bf16)] * 4
        + [jax.ShapeDtypeStruct((_NF, 128), f32)] * 2,
        interpret=_INTERPRET,
    )
    prwh, prwl, piwh, piwl, pbr, pbi = fwdpw(
        pq, pk, jnp.asarray(fch), jnp.asarray(fcl),
        jnp.asarray(fsh), jnp.asarray(fsl),
        jnp.asarray(wrow.reshape(_NF, 1)), jnp.asarray(gfull))

    TB = 512
    inv2 = pl.pallas_call(
        _inv2_body,
        grid=(B, L // TB),
        in_specs=[pl.BlockSpec((1, D, _NF), lambda b, j: (b, 0, 0))] * 4
        + [pl.BlockSpec((_NF, TB), lambda b, j: (0, j))] * 4,
        out_specs=pl.BlockSpec((1, D, TB), lambda b, j: (b, 0, j)),
        out_shape=jax.ShapeDtypeStruct((B, D, L), f32),
        interpret=_INTERPRET,
    )
    corr = inv2(prwh, prwl, piwh, piwl,
                jnp.asarray(fch), jnp.asarray(fcl),
                jnp.asarray(fsh), jnp.asarray(fsl))

    mv = pl.pallas_call(
        _mv_body,
        grid=(1,),
        in_specs=[pl.BlockSpec((L, _NF), lambda i: (0, 0))] * 2
        + [pl.BlockSpec((_NF, 128), lambda i: (0, 0))] * 2,
        out_specs=[pl.BlockSpec((L, 128), lambda i: (0, 0)),
                   pl.BlockSpec((L, 1), lambda i: (0, 0))],
        out_shape=[jax.ShapeDtypeStruct((L, 128), f32),
                   jax.ShapeDtypeStruct((L, 1), f32)],
        interpret=_INTERPRET,
    )
    mvt64, mcol = mv(jnp.asarray(wct), jnp.asarray(wst), pbr, pbi)

    idx16, smt_flat = _make_topk_sc()(mcol.reshape(L), mvt64)
    smw = jnp.tile(smt_flat.reshape(8, _DK), (1, _H))

    HCB2 = 256
    agg = pl.pallas_call(
        _agg_body,
        grid=(B, D // HCB2),
        in_specs=[pl.BlockSpec(memory_space=pltpu.SMEM),
                  pl.BlockSpec((1, L, HCB2), lambda b, h: (b, 0, h)),
                  pl.BlockSpec((8, HCB2), lambda b, h: (0, h)),
                  pl.BlockSpec((HCB2, D), lambda b, h: (h, 0)),
                  pl.BlockSpec((1, D), lambda b, h: (0, 0))],
        out_specs=pl.BlockSpec((1, L, D), lambda b, h: (b, 0, 0)),
        out_shape=jax.ShapeDtypeStruct((B, L, D), f32),
        interpret=_INTERPRET,
    )
    out = agg(idx16, pv, smw, Wo.T, bo.reshape(1, D))

    return out, corr.reshape(B, _H, _DK, L)
